# Initial kernel scaffold; baseline (speedup 1.0000x reference)
import functools
import jax
import jax.numpy as jnp
from jax.experimental import pallas as pl
from jax.experimental.pallas import tpu as pltpu

N_GENE = 10000
NUM_DRUGS = 8000
NUM_DISEASES = 20000
HID = 128
HEADS = 4
BLK = 1000


def _wprep_body(wsd_ref, wdd_ref, asd_ref, add_ref,
                wss_ref, wds_ref, ass_ref, ads_ref, v_ref):
    cols = []
    for w_ref, a_ref in ((wsd_ref, asd_ref), (wdd_ref, add_ref),
                         (wss_ref, ass_ref), (wds_ref, ads_ref)):
        w = w_ref[...]
        a = a_ref[...]
        for h in range(HEADS):
            col = jnp.dot(w[:, h * HID:(h + 1) * HID], a[h, :],
                          preferred_element_type=jnp.float32)
            cols.append(col[:, None])
    v_ref[...] = jnp.concatenate(cols, axis=1)


def _weights_prep(wsd, asd, wdd, add, wss, ass, wds, ads):
    return pl.pallas_call(
        _wprep_body,
        out_shape=jax.ShapeDtypeStruct((HID, 16), jnp.float32),
    )(wsd, wdd, asd, add, wss, wds, ass, ads)


def _h1_body(x_ref, w_ref, deg_ref, h1_ref, h1p_ref, dinv_ref):
    h1 = jnp.dot(x_ref[...], w_ref[...], preferred_element_type=jnp.float32)
    dinv = jax.lax.rsqrt(deg_ref[...] + 1.0)
    h1_ref[...] = h1
    h1p_ref[...] = dinv * h1
    dinv_ref[...] = dinv


def _h1_stage(gene_nodes, gcn1_W, deg2d):
    g = N_GENE // BLK
    return pl.pallas_call(
        _h1_body,
        grid=(g,),
        in_specs=[pl.BlockSpec((BLK, HID), lambda i: (i, 0)),
                  pl.BlockSpec((HID, HID), lambda i: (0, 0)),
                  pl.BlockSpec((BLK, 1), lambda i: (i, 0))],
        out_specs=[pl.BlockSpec((BLK, HID), lambda i: (i, 0)),
                   pl.BlockSpec((BLK, HID), lambda i: (i, 0)),
                   pl.BlockSpec((BLK, 1), lambda i: (i, 0))],
        out_shape=[jax.ShapeDtypeStruct((N_GENE, HID), jnp.float32),
                   jax.ShapeDtypeStruct((N_GENE, HID), jnp.float32),
                   jax.ShapeDtypeStruct((N_GENE, 1), jnp.float32)],
    )(gene_nodes, gcn1_W, deg2d)


def _mid_body(s_ref, h_ref, dinv_ref, b_ref, w2_ref, h2_ref, h2p_ref):
    dinv = dinv_ref[...]
    x1 = jax.nn.relu(dinv * s_ref[...] + dinv * dinv * h_ref[...] + b_ref[...])
    h2 = jnp.dot(x1, w2_ref[...], preferred_element_type=jnp.float32)
    h2_ref[...] = h2
    h2p_ref[...] = dinv * h2


def _mid_stage(s1, h1, dinv, b1, gcn2_W):
    g = N_GENE // BLK
    return pl.pallas_call(
        _mid_body,
        grid=(g,),
        in_specs=[pl.BlockSpec((BLK, HID), lambda i: (i, 0)),
                  pl.BlockSpec((BLK, HID), lambda i: (i, 0)),
                  pl.BlockSpec((BLK, 1), lambda i: (i, 0)),
                  pl.BlockSpec((1, HID), lambda i: (0, 0)),
                  pl.BlockSpec((HID, HID), lambda i: (0, 0))],
        out_specs=[pl.BlockSpec((BLK, HID), lambda i: (i, 0)),
                   pl.BlockSpec((BLK, HID), lambda i: (i, 0))],
        out_shape=[jax.ShapeDtypeStruct((N_GENE, HID), jnp.float32),
                   jax.ShapeDtypeStruct((N_GENE, HID), jnp.float32)],
    )(s1, h1, dinv, b1, gcn2_W)


def _post_body(s_ref, h_ref, dinv_ref, b_ref, vd_ref, adst_ref):
    dinv = dinv_ref[...]
    x = jax.nn.relu(dinv * s_ref[...] + dinv * dinv * h_ref[...] + b_ref[...])
    adst_ref[...] = jnp.dot(x, vd_ref[...], preferred_element_type=jnp.float32)


def _post_stage(s2, h2, dinv, b2, vdst):
    g = N_GENE // BLK
    return pl.pallas_call(
        _post_body,
        grid=(g,),
        in_specs=[pl.BlockSpec((BLK, HID), lambda i: (i, 0)),
                  pl.BlockSpec((BLK, HID), lambda i: (i, 0)),
                  pl.BlockSpec((BLK, 1), lambda i: (i, 0)),
                  pl.BlockSpec((1, HID), lambda i: (0, 0)),
                  pl.BlockSpec((HID, 8), lambda i: (0, 0))],
        out_specs=pl.BlockSpec((BLK, 8), lambda i: (i, 0)),
        out_shape=jax.ShapeDtypeStruct((N_GENE, 8), jnp.float32),
    )(s2, h2, dinv, b2, vdst)


def _table_body(emb_ref, vs_ref, asrc_ref, colsum_ref):
    emb = emb_ref[...]
    asrc_ref[...] = jnp.dot(emb, vs_ref[...], preferred_element_type=jnp.float32)
    colsum_ref[...] = jnp.sum(emb, axis=0, keepdims=True)


def _table_stage(emb, vs):
    n = emb.shape[0]
    g = n // BLK
    return pl.pallas_call(
        _table_body,
        grid=(g,),
        in_specs=[pl.BlockSpec((BLK, HID), lambda i: (i, 0)),
                  pl.BlockSpec((HID, HEADS), lambda i: (0, 0))],
        out_specs=[pl.BlockSpec((BLK, HEADS), lambda i: (i, 0)),
                   pl.BlockSpec((1, HID), lambda i: (i, 0))],
        out_shape=[jax.ShapeDtypeStruct((n, HEADS), jnp.float32),
                   jax.ShapeDtypeStruct((g, HID), jnp.float32)],
    )(emb, vs)


def _emax_body(adst_ref, asd_ref, ass_ref, e_ref):
    md = jnp.max(asd_ref[...], axis=0)
    ms = jnp.max(ass_ref[...], axis=0)
    add_ = jnp.max(adst_ref[...], axis=0)
    raw = jnp.concatenate([md + add_[:4], ms + add_[4:]])
    e_ref[...] = jnp.maximum(raw, 0.2 * raw)[None, :]


def _emax_stage(adst, asrc_d, asrc_s):
    return pl.pallas_call(
        _emax_body,
        out_shape=jax.ShapeDtypeStruct((1, 8), jnp.float32),
    )(adst, asrc_d, asrc_s)


def _u_body(w_ref, emb_ref, u_ref):
    @pl.when(pl.program_id(0) == 0)
    def _():
        u_ref[...] = jnp.zeros_like(u_ref)
    u_ref[...] += jax.lax.dot_general(
        w_ref[...], emb_ref[...], (((0,), (0,)), ((), ())),
        preferred_element_type=jnp.float32)


def _u_stage(w, emb):
    n = w.shape[0]
    g = n // BLK
    return pl.pallas_call(
        _u_body,
        grid=(g,),
        in_specs=[pl.BlockSpec((BLK, HEADS), lambda i: (i, 0)),
                  pl.BlockSpec((BLK, HID), lambda i: (i, 0))],
        out_specs=pl.BlockSpec((HEADS, HID), lambda i: (0, 0)),
        out_shape=jax.ShapeDtypeStruct((HEADS, HID), jnp.float32),
    )(w, emb)


def _final_body(ud_ref, us_ref, wd_ref, ws_ref, bd_ref, bs_ref,
                csd_ref, css_ref, fw_ref, fb_ref, r_ref):
    scale = 1.0 / (N_GENE * HEADS)
    md = jnp.zeros((1, HID), jnp.float32)
    ms = jnp.zeros((1, HID), jnp.float32)
    for h in range(HEADS):
        md += jnp.dot(ud_ref[h, :][None, :], wd_ref[:, h * HID:(h + 1) * HID],
                      preferred_element_type=jnp.float32)
        ms += jnp.dot(us_ref[h, :][None, :], ws_ref[:, h * HID:(h + 1) * HID],
                      preferred_element_type=jnp.float32)
    md = md * scale + bd_ref[...]
    ms = ms * scale + bs_ref[...]
    bg = 0.5 * (md + ms)
    bdm = jnp.sum(csd_ref[...], axis=0, keepdims=True) / NUM_DRUGS
    bcm = jnp.sum(css_ref[...], axis=0, keepdims=True) / NUM_DISEASES
    cat = jnp.concatenate([bg, bdm, bcm], axis=1)
    r_ref[...] = jnp.dot(cat, fw_ref[...],
                         preferred_element_type=jnp.float32) + fb_ref[...]


def _final_stage(u_d, u_s, wsrc_d, wsrc_s, b_d, b_s, cs_d, cs_s, fuse_W, fuse_b):
    return pl.pallas_call(
        _final_body,
        out_shape=jax.ShapeDtypeStruct((1, HID), jnp.float32),
    )(u_d, u_s, wsrc_d, wsrc_s, b_d[None, :], b_s[None, :], cs_d, cs_s,
      fuse_W, fuse_b[None, :])


def _deg_stage(gene_dst):
    return jnp.zeros((N_GENE,), jnp.float32).at[gene_dst].add(1.0)


def _segsum_stage(hp, src, dst):
    return jnp.zeros((N_GENE, HID), jnp.float32).at[dst].add(hp[src])


def _gat_edge_stage(asrc, adst, emax, src, dst, n_src):
    e = asrc[src] + adst[dst]
    e = jnp.maximum(e, 0.2 * e)
    ex = jnp.exp(e - emax[None, :])
    denom = jnp.zeros((N_GENE, HEADS), jnp.float32).at[dst].add(ex)
    alpha = ex / (denom[dst] + 1e-16)
    return jnp.zeros((n_src, HEADS), jnp.float32).at[src].add(alpha)


def kernel(gene_nodes, drug_edges, disease_edges, gene_edges,
           gcn1_W, gcn1_b, gcn2_W, gcn2_b, drug_emb, dis_emb,
           gat_d_Wsrc, gat_d_Wdst, gat_d_asrc, gat_d_adst, gat_d_b,
           gat_s_Wsrc, gat_s_Wdst, gat_s_asrc, gat_s_adst, gat_s_b,
           fuse_W, fuse_b):
    v = _weights_prep(gat_d_Wsrc, gat_d_asrc, gat_d_Wdst, gat_d_adst,
                      gat_s_Wsrc, gat_s_asrc, gat_s_Wdst, gat_s_adst)
    vsd, vdd, vss, vds = v[:, 0:4], v[:, 4:8], v[:, 8:12], v[:, 12:16]
    vdst = jnp.concatenate([vdd, vds], axis=1)

    gsrc, gdst = gene_edges[0], gene_edges[1]
    deg = _deg_stage(gdst)
    h1, h1p, dinv = _h1_stage(gene_nodes, gcn1_W, deg[:, None])
    s1 = _segsum_stage(h1p, gsrc, gdst)
    h2, h2p = _mid_stage(s1, h1, dinv, gcn1_b[None, :], gcn2_W)
    s2 = _segsum_stage(h2p, gsrc, gdst)
    adst = _post_stage(s2, h2, dinv, gcn2_b[None, :], vdst)

    asrc_d, cs_d = _table_stage(drug_emb, vsd)
    asrc_s, cs_s = _table_stage(dis_emb, vss)
    emax = _emax_stage(adst, asrc_d, asrc_s)[0]

    w_d = _gat_edge_stage(asrc_d, adst[:, :4], emax[:4],
                          drug_edges[0], drug_edges[1], NUM_DRUGS)
    w_s = _gat_edge_stage(asrc_s[:N_GENE], adst[:, 4:], emax[4:],
                          disease_edges[0], disease_edges[1], N_GENE)

    u_d = _u_stage(w_d, drug_emb)
    u_s = _u_stage(w_s, dis_emb[:N_GENE])

    return _final_stage(u_d, u_s, gat_d_Wsrc, gat_s_Wsrc, gat_d_b, gat_s_b,
                        cs_d, cs_s, fuse_W, fuse_b)


# speedup vs baseline: 7.2544x; 7.2544x over previous
import functools
import jax
import jax.numpy as jnp
from jax.experimental import pallas as pl
from jax.experimental.pallas import tpu as pltpu

N_GENE = 10000
NUM_DRUGS = 8000
NUM_DISEASES = 20000
HID = 128
HEADS = 4
BLK = 1000


def _wprep_body(wsd_ref, wdd_ref, asd_ref, add_ref,
                wss_ref, wds_ref, ass_ref, ads_ref, v_ref):
    cols = []
    for w_ref, a_ref in ((wsd_ref, asd_ref), (wdd_ref, add_ref),
                         (wss_ref, ass_ref), (wds_ref, ads_ref)):
        w = w_ref[...]
        a = a_ref[...]
        for h in range(HEADS):
            col = jnp.dot(w[:, h * HID:(h + 1) * HID], a[h, :],
                          preferred_element_type=jnp.float32)
            cols.append(col[:, None])
    v_ref[...] = jnp.concatenate(cols, axis=1)


def _weights_prep(wsd, asd, wdd, add, wss, ass, wds, ads):
    return pl.pallas_call(
        _wprep_body,
        out_shape=jax.ShapeDtypeStruct((HID, 16), jnp.float32),
    )(wsd, wdd, asd, add, wss, wds, ass, ads)


def _h1_body(x_ref, w_ref, deg_ref, h1_ref, h1p_ref, dinv_ref):
    h1 = jnp.dot(x_ref[...], w_ref[...], preferred_element_type=jnp.float32)
    dinv = jax.lax.rsqrt(deg_ref[...] + 1.0)
    h1_ref[...] = h1
    h1p_ref[...] = dinv * h1
    dinv_ref[...] = dinv


def _h1_stage(gene_nodes, gcn1_W, deg2d):
    g = N_GENE // BLK
    return pl.pallas_call(
        _h1_body,
        grid=(g,),
        in_specs=[pl.BlockSpec((BLK, HID), lambda i: (i, 0)),
                  pl.BlockSpec((HID, HID), lambda i: (0, 0)),
                  pl.BlockSpec((BLK, 1), lambda i: (i, 0))],
        out_specs=[pl.BlockSpec((BLK, HID), lambda i: (i, 0)),
                   pl.BlockSpec((BLK, HID), lambda i: (i, 0)),
                   pl.BlockSpec((BLK, 1), lambda i: (i, 0))],
        out_shape=[jax.ShapeDtypeStruct((N_GENE, HID), jnp.float32),
                   jax.ShapeDtypeStruct((N_GENE, HID), jnp.float32),
                   jax.ShapeDtypeStruct((N_GENE, 1), jnp.float32)],
    )(gene_nodes, gcn1_W, deg2d)


def _mid_body(s_ref, h_ref, dinv_ref, b_ref, w2_ref, h2_ref, h2p_ref):
    dinv = dinv_ref[...]
    x1 = jax.nn.relu(dinv * s_ref[...] + dinv * dinv * h_ref[...] + b_ref[...])
    h2 = jnp.dot(x1, w2_ref[...], preferred_element_type=jnp.float32)
    h2_ref[...] = h2
    h2p_ref[...] = dinv * h2


def _mid_stage(s1, h1, dinv, b1, gcn2_W):
    g = N_GENE // BLK
    return pl.pallas_call(
        _mid_body,
        grid=(g,),
        in_specs=[pl.BlockSpec((BLK, HID), lambda i: (i, 0)),
                  pl.BlockSpec((BLK, HID), lambda i: (i, 0)),
                  pl.BlockSpec((BLK, 1), lambda i: (i, 0)),
                  pl.BlockSpec((1, HID), lambda i: (0, 0)),
                  pl.BlockSpec((HID, HID), lambda i: (0, 0))],
        out_specs=[pl.BlockSpec((BLK, HID), lambda i: (i, 0)),
                   pl.BlockSpec((BLK, HID), lambda i: (i, 0))],
        out_shape=[jax.ShapeDtypeStruct((N_GENE, HID), jnp.float32),
                   jax.ShapeDtypeStruct((N_GENE, HID), jnp.float32)],
    )(s1, h1, dinv, b1, gcn2_W)


def _post_body(s_ref, h_ref, dinv_ref, b_ref, vd_ref, adst_ref):
    dinv = dinv_ref[...]
    x = jax.nn.relu(dinv * s_ref[...] + dinv * dinv * h_ref[...] + b_ref[...])
    adst_ref[...] = jnp.dot(x, vd_ref[...], preferred_element_type=jnp.float32)


def _post_stage(s2, h2, dinv, b2, vdst):
    g = N_GENE // BLK
    return pl.pallas_call(
        _post_body,
        grid=(g,),
        in_specs=[pl.BlockSpec((BLK, HID), lambda i: (i, 0)),
                  pl.BlockSpec((BLK, HID), lambda i: (i, 0)),
                  pl.BlockSpec((BLK, 1), lambda i: (i, 0)),
                  pl.BlockSpec((1, HID), lambda i: (0, 0)),
                  pl.BlockSpec((HID, 8), lambda i: (0, 0))],
        out_specs=pl.BlockSpec((BLK, 8), lambda i: (i, 0)),
        out_shape=jax.ShapeDtypeStruct((N_GENE, 8), jnp.float32),
    )(s2, h2, dinv, b2, vdst)


def _table_body(emb_ref, vs_ref, asrc_ref, colsum_ref):
    emb = emb_ref[...]
    asrc_ref[...] = jnp.dot(emb, vs_ref[...], preferred_element_type=jnp.float32)
    colsum_ref[...] = jnp.sum(emb, axis=0, keepdims=True)[None]


def _table_stage(emb, vs):
    n = emb.shape[0]
    g = n // BLK
    return pl.pallas_call(
        _table_body,
        grid=(g,),
        in_specs=[pl.BlockSpec((BLK, HID), lambda i: (i, 0)),
                  pl.BlockSpec((HID, HEADS), lambda i: (0, 0))],
        out_specs=[pl.BlockSpec((BLK, HEADS), lambda i: (i, 0)),
                   pl.BlockSpec((1, 1, HID), lambda i: (i, 0, 0))],
        out_shape=[jax.ShapeDtypeStruct((n, HEADS), jnp.float32),
                   jax.ShapeDtypeStruct((g, 1, HID), jnp.float32)],
    )(emb, vs)


def _emax_body(adst_ref, asd_ref, ass_ref, e_ref):
    md = jnp.max(asd_ref[...], axis=0)
    ms = jnp.max(ass_ref[...], axis=0)
    add_ = jnp.max(adst_ref[...], axis=0)
    raw = jnp.concatenate([md + add_[:4], ms + add_[4:]])
    e_ref[...] = jnp.maximum(raw, 0.2 * raw)[None, :]


def _emax_stage(adst, asrc_d, asrc_s):
    return pl.pallas_call(
        _emax_body,
        out_shape=jax.ShapeDtypeStruct((1, 8), jnp.float32),
    )(adst, asrc_d, asrc_s)


def _u_body(w_ref, emb_ref, u_ref):
    @pl.when(pl.program_id(0) == 0)
    def _():
        u_ref[...] = jnp.zeros_like(u_ref)
    u_ref[...] += jax.lax.dot_general(
        w_ref[...], emb_ref[...], (((0,), (0,)), ((), ())),
        preferred_element_type=jnp.float32)


def _u_stage(w, emb):
    n = w.shape[0]
    g = n // BLK
    return pl.pallas_call(
        _u_body,
        grid=(g,),
        in_specs=[pl.BlockSpec((BLK, HEADS), lambda i: (i, 0)),
                  pl.BlockSpec((BLK, HID), lambda i: (i, 0))],
        out_specs=pl.BlockSpec((HEADS, HID), lambda i: (0, 0)),
        out_shape=jax.ShapeDtypeStruct((HEADS, HID), jnp.float32),
    )(w, emb)


def _final_body(ud_ref, us_ref, wd_ref, ws_ref, bd_ref, bs_ref,
                csd_ref, css_ref, fw_ref, fb_ref, r_ref):
    scale = 1.0 / (N_GENE * HEADS)
    md = jnp.zeros((1, HID), jnp.float32)
    ms = jnp.zeros((1, HID), jnp.float32)
    for h in range(HEADS):
        md += jnp.dot(ud_ref[h, :][None, :], wd_ref[:, h * HID:(h + 1) * HID],
                      preferred_element_type=jnp.float32)
        ms += jnp.dot(us_ref[h, :][None, :], ws_ref[:, h * HID:(h + 1) * HID],
                      preferred_element_type=jnp.float32)
    md = md * scale + bd_ref[...]
    ms = ms * scale + bs_ref[...]
    bg = 0.5 * (md + ms)
    bdm = jnp.sum(csd_ref[...], axis=0, keepdims=True) / NUM_DRUGS
    bcm = jnp.sum(css_ref[...], axis=0, keepdims=True) / NUM_DISEASES
    cat = jnp.concatenate([bg, bdm, bcm], axis=1)
    r_ref[...] = jnp.dot(cat, fw_ref[...],
                         preferred_element_type=jnp.float32) + fb_ref[...]


def _final_stage(u_d, u_s, wsrc_d, wsrc_s, b_d, b_s, cs_d, cs_s, fuse_W, fuse_b):
    return pl.pallas_call(
        _final_body,
        out_shape=jax.ShapeDtypeStruct((1, HID), jnp.float32),
    )(u_d, u_s, wsrc_d, wsrc_s, b_d[None, :], b_s[None, :], cs_d, cs_s,
      fuse_W, fuse_b[None, :])


def _deg_stage(gene_dst):
    return jnp.zeros((N_GENE,), jnp.float32).at[gene_dst].add(1.0)


def _segsum_stage(hp, src, dst):
    return jnp.zeros((N_GENE, HID), jnp.float32).at[dst].add(hp[src])


def _gat_edge_stage(asrc, adst, emax, src, dst, n_src):
    e = asrc[src] + adst[dst]
    e = jnp.maximum(e, 0.2 * e)
    ex = jnp.exp(e - emax[None, :])
    denom = jnp.zeros((N_GENE, HEADS), jnp.float32).at[dst].add(ex)
    alpha = ex / (denom[dst] + 1e-16)
    return jnp.zeros((n_src, HEADS), jnp.float32).at[src].add(alpha)


def kernel(gene_nodes, drug_edges, disease_edges, gene_edges,
           gcn1_W, gcn1_b, gcn2_W, gcn2_b, drug_emb, dis_emb,
           gat_d_Wsrc, gat_d_Wdst, gat_d_asrc, gat_d_adst, gat_d_b,
           gat_s_Wsrc, gat_s_Wdst, gat_s_asrc, gat_s_adst, gat_s_b,
           fuse_W, fuse_b):
    v = _weights_prep(gat_d_Wsrc, gat_d_asrc, gat_d_Wdst, gat_d_adst,
                      gat_s_Wsrc, gat_s_asrc, gat_s_Wdst, gat_s_adst)
    vsd, vdd, vss, vds = v[:, 0:4], v[:, 4:8], v[:, 8:12], v[:, 12:16]
    vdst = jnp.concatenate([vdd, vds], axis=1)

    gsrc, gdst = gene_edges[0], gene_edges[1]
    deg = _deg_stage(gdst)
    h1, h1p, dinv = _h1_stage(gene_nodes, gcn1_W, deg[:, None])
    s1 = _segsum_stage(h1p, gsrc, gdst)
    h2, h2p = _mid_stage(s1, h1, dinv, gcn1_b[None, :], gcn2_W)
    s2 = _segsum_stage(h2p, gsrc, gdst)
    adst = _post_stage(s2, h2, dinv, gcn2_b[None, :], vdst)

    asrc_d, cs_d = _table_stage(drug_emb, vsd)
    asrc_s, cs_s = _table_stage(dis_emb, vss)
    emax = _emax_stage(adst, asrc_d, asrc_s)[0]

    w_d = _gat_edge_stage(asrc_d, adst[:, :4], emax[:4],
                          drug_edges[0], drug_edges[1], NUM_DRUGS)
    w_s = _gat_edge_stage(asrc_s[:N_GENE], adst[:, 4:], emax[4:],
                          disease_edges[0], disease_edges[1], N_GENE)

    u_d = _u_stage(w_d, drug_emb)
    u_s = _u_stage(w_s, dis_emb[:N_GENE])

    return _final_stage(u_d, u_s, gat_d_Wsrc, gat_s_Wsrc, gat_d_b, gat_s_b,
                        cs_d[:, 0, :], cs_s[:, 0, :], fuse_W, fuse_b)


# speedup vs baseline: 34.0582x; 4.6948x over previous
import functools
import jax
import jax.numpy as jnp
from jax import lax
from jax.experimental import pallas as pl
from jax.experimental.pallas import tpu as pltpu
from jax.experimental.pallas import tpu_sc as plsc

N_GENE = 10000
NUM_DRUGS = 8000
NUM_DISEASES = 20000
HID = 128
HEADS = 4
BLK = 1000


def _wprep_body(wsd_ref, wdd_ref, asd_ref, add_ref,
                wss_ref, wds_ref, ass_ref, ads_ref, v_ref):
    cols = []
    for w_ref, a_ref in ((wsd_ref, asd_ref), (wdd_ref, add_ref),
                         (wss_ref, ass_ref), (wds_ref, ads_ref)):
        w = w_ref[...]
        a = a_ref[...]
        for h in range(HEADS):
            col = jnp.dot(w[:, h * HID:(h + 1) * HID], a[h, :],
                          preferred_element_type=jnp.float32)
            cols.append(col[:, None])
    v_ref[...] = jnp.concatenate(cols, axis=1)


def _weights_prep(wsd, asd, wdd, add, wss, ass, wds, ads):
    return pl.pallas_call(
        _wprep_body,
        out_shape=jax.ShapeDtypeStruct((HID, 16), jnp.float32),
    )(wsd, wdd, asd, add, wss, wds, ass, ads)


def _h1_body(x_ref, w_ref, deg_ref, h1_ref, h1p_ref, dinv_ref):
    h1 = jnp.dot(x_ref[...], w_ref[...], preferred_element_type=jnp.float32)
    dinv = jax.lax.rsqrt(deg_ref[0] + deg_ref[1] + 1.0)
    h1_ref[...] = h1
    h1p_ref[...] = dinv * h1
    dinv_ref[...] = dinv


def _h1_stage(gene_nodes, gcn1_W, deg2d):
    g = N_GENE // BLK
    return pl.pallas_call(
        _h1_body,
        grid=(g,),
        in_specs=[pl.BlockSpec((BLK, HID), lambda i: (i, 0)),
                  pl.BlockSpec((HID, HID), lambda i: (0, 0)),
                  pl.BlockSpec((2, BLK, 1), lambda i: (0, i, 0))],
        out_specs=[pl.BlockSpec((BLK, HID), lambda i: (i, 0)),
                   pl.BlockSpec((BLK, HID), lambda i: (i, 0)),
                   pl.BlockSpec((BLK, 1), lambda i: (i, 0))],
        out_shape=[jax.ShapeDtypeStruct((N_GENE, HID), jnp.float32),
                   jax.ShapeDtypeStruct((N_GENE, HID), jnp.float32),
                   jax.ShapeDtypeStruct((N_GENE, 1), jnp.float32)],
    )(gene_nodes, gcn1_W, deg2d)


def _mid_body(s_ref, h_ref, dinv_ref, b_ref, w2_ref, h2_ref, h2p_ref):
    dinv = dinv_ref[...]
    x1 = jax.nn.relu(dinv * (s_ref[0] + s_ref[1]) + dinv * dinv * h_ref[...]
                     + b_ref[...])
    h2 = jnp.dot(x1, w2_ref[...], preferred_element_type=jnp.float32)
    h2_ref[...] = h2
    h2p_ref[...] = dinv * h2


def _mid_stage(s1, h1, dinv, b1, gcn2_W):
    g = N_GENE // BLK
    return pl.pallas_call(
        _mid_body,
        grid=(g,),
        in_specs=[pl.BlockSpec((2, BLK, HID), lambda i: (0, i, 0)),
                  pl.BlockSpec((BLK, HID), lambda i: (i, 0)),
                  pl.BlockSpec((BLK, 1), lambda i: (i, 0)),
                  pl.BlockSpec((1, HID), lambda i: (0, 0)),
                  pl.BlockSpec((HID, HID), lambda i: (0, 0))],
        out_specs=[pl.BlockSpec((BLK, HID), lambda i: (i, 0)),
                   pl.BlockSpec((BLK, HID), lambda i: (i, 0))],
        out_shape=[jax.ShapeDtypeStruct((N_GENE, HID), jnp.float32),
                   jax.ShapeDtypeStruct((N_GENE, HID), jnp.float32)],
    )(s1, h1, dinv, b1, gcn2_W)


def _post_body(s_ref, h_ref, dinv_ref, b_ref, vd_ref, adst_ref):
    dinv = dinv_ref[...]
    x = jax.nn.relu(dinv * (s_ref[0] + s_ref[1]) + dinv * dinv * h_ref[...]
                    + b_ref[...])
    adst_ref[...] = jnp.dot(x, vd_ref[...], preferred_element_type=jnp.float32)


def _post_stage(s2, h2, dinv, b2, vdst):
    g = N_GENE // BLK
    return pl.pallas_call(
        _post_body,
        grid=(g,),
        in_specs=[pl.BlockSpec((2, BLK, HID), lambda i: (0, i, 0)),
                  pl.BlockSpec((BLK, HID), lambda i: (i, 0)),
                  pl.BlockSpec((BLK, 1), lambda i: (i, 0)),
                  pl.BlockSpec((1, HID), lambda i: (0, 0)),
                  pl.BlockSpec((HID, 8), lambda i: (0, 0))],
        out_specs=pl.BlockSpec((BLK, 8), lambda i: (i, 0)),
        out_shape=jax.ShapeDtypeStruct((N_GENE, 8), jnp.float32),
    )(s2, h2, dinv, b2, vdst)


def _table_body(emb_ref, vs_ref, asrc_ref, colsum_ref):
    emb = emb_ref[...]
    asrc_ref[...] = jnp.dot(emb, vs_ref[...], preferred_element_type=jnp.float32)
    colsum_ref[...] = jnp.sum(emb, axis=0, keepdims=True)[None]


def _table_stage(emb, vs):
    n = emb.shape[0]
    g = n // BLK
    return pl.pallas_call(
        _table_body,
        grid=(g,),
        in_specs=[pl.BlockSpec((BLK, HID), lambda i: (i, 0)),
                  pl.BlockSpec((HID, HEADS), lambda i: (0, 0))],
        out_specs=[pl.BlockSpec((BLK, HEADS), lambda i: (i, 0)),
                   pl.BlockSpec((1, 1, HID), lambda i: (i, 0, 0))],
        out_shape=[jax.ShapeDtypeStruct((n, HEADS), jnp.float32),
                   jax.ShapeDtypeStruct((g, 1, HID), jnp.float32)],
    )(emb, vs)


def _emax_body(adst_ref, asd_ref, ass_ref, e_ref):
    md = jnp.max(asd_ref[...], axis=0)
    ms = jnp.max(ass_ref[...], axis=0)
    add_ = jnp.max(adst_ref[...], axis=0)
    raw = jnp.concatenate([md + add_[:4], ms + add_[4:]])
    e_ref[...] = jnp.maximum(raw, 0.2 * raw)[None, :]


def _emax_stage(adst, asrc_d, asrc_s):
    return pl.pallas_call(
        _emax_body,
        out_shape=jax.ShapeDtypeStruct((1, 8), jnp.float32),
    )(adst, asrc_d, asrc_s)


def _psum_body(a_ref, o_ref):
    o_ref[...] = a_ref[0] + a_ref[1]


def _psum_stage(p):
    n = p.shape[1]
    return pl.pallas_call(
        _psum_body,
        out_shape=jax.ShapeDtypeStruct((n, 4), jnp.float32),
    )(p)


def _u_body(w_ref, emb_ref, u_ref):
    @pl.when(pl.program_id(0) == 0)
    def _():
        u_ref[...] = jnp.zeros_like(u_ref)
    u_ref[...] += jax.lax.dot_general(
        w_ref[0] + w_ref[1], emb_ref[...], (((0,), (0,)), ((), ())),
        preferred_element_type=jnp.float32)


def _u_stage(w, emb):
    n = w.shape[1]
    g = n // BLK
    return pl.pallas_call(
        _u_body,
        grid=(g,),
        in_specs=[pl.BlockSpec((2, BLK, HEADS), lambda i: (0, i, 0)),
                  pl.BlockSpec((BLK, HID), lambda i: (i, 0))],
        out_specs=pl.BlockSpec((HEADS, HID), lambda i: (0, 0)),
        out_shape=jax.ShapeDtypeStruct((HEADS, HID), jnp.float32),
    )(w, emb)


def _final_body(ud_ref, us_ref, wd_ref, ws_ref, bd_ref, bs_ref,
                csd_ref, css_ref, fw_ref, fb_ref, r_ref):
    scale = 1.0 / (N_GENE * HEADS)
    md = jnp.zeros((1, HID), jnp.float32)
    ms = jnp.zeros((1, HID), jnp.float32)
    for h in range(HEADS):
        md += jnp.dot(ud_ref[h, :][None, :], wd_ref[:, h * HID:(h + 1) * HID],
                      preferred_element_type=jnp.float32)
        ms += jnp.dot(us_ref[h, :][None, :], ws_ref[:, h * HID:(h + 1) * HID],
                      preferred_element_type=jnp.float32)
    md = md * scale + bd_ref[...]
    ms = ms * scale + bs_ref[...]
    bg = 0.5 * (md + ms)
    bdm = jnp.sum(csd_ref[...], axis=0, keepdims=True) / NUM_DRUGS
    bcm = jnp.sum(css_ref[...], axis=0, keepdims=True) / NUM_DISEASES
    cat = jnp.concatenate([bg, bdm, bcm], axis=1)
    r_ref[...] = jnp.dot(cat, fw_ref[...],
                         preferred_element_type=jnp.float32) + fb_ref[...]


def _final_stage(u_d, u_s, wsrc_d, wsrc_s, b_d, b_s, cs_d, cs_s, fuse_W, fuse_b):
    return pl.pallas_call(
        _final_body,
        out_shape=jax.ShapeDtypeStruct((1, HID), jnp.float32),
    )(u_d, u_s, wsrc_d, wsrc_s, b_d[None, :], b_s[None, :], cs_d, cs_s,
      fuse_W, fuse_b[None, :])


_MESH = plsc.VectorSubcoreMesh(core_axis_name="c", subcore_axis_name="s")
NCORE = 2
NSUB = 16
GCHUNK = 80
GCN_CH = 125
GAT_E = 204800
GAT_CH = 80
NPAD = 10112
DPAD = 8192


def _sc_deg(dst3, ones_rows, zeros_nodes):
    @functools.partial(
        pl.kernel,
        out_type=jax.ShapeDtypeStruct((NCORE, NPAD, 4), jnp.float32),
        mesh=_MESH,
        scratch_types=[
            pltpu.VMEM((GCN_CH, GCHUNK), jnp.int32),
            pltpu.VMEM((GCHUNK, 4), jnp.float32),
            pltpu.VMEM_SHARED((NPAD, 4), jnp.float32),
        ],
    )
    def k(dst_h, ones_h, zeros_h, out_h, idx_v, ones_v, acc_sh):
        c = lax.axis_index("c")
        s = lax.axis_index("s")
        rows = NPAD // NSUB
        pltpu.sync_copy(zeros_h.at[pl.ds(0, rows)], acc_sh.at[pl.ds(s * rows, rows)])
        plsc.subcore_barrier()
        pltpu.sync_copy(dst_h.at[c * NSUB + s], idx_v)
        pltpu.sync_copy(ones_h, ones_v)

        def body(j, _):
            pltpu.sync_copy(ones_v, acc_sh.at[idx_v.at[j]], add=True)
            return 0

        lax.fori_loop(0, GCN_CH, body, 0)
        plsc.subcore_barrier()
        pltpu.sync_copy(acc_sh.at[pl.ds(s * rows, rows)],
                        out_h.at[c, pl.ds(s * rows, rows)])

    return k(dst3, ones_rows, zeros_nodes)


def _sc_segsum(hp, src3, dst3, zeros_rows):
    @functools.partial(
        pl.kernel,
        out_type=jax.ShapeDtypeStruct((NCORE, NPAD, HID), jnp.float32),
        mesh=_MESH,
        scratch_types=[
            pltpu.VMEM((GCN_CH, GCHUNK), jnp.int32),
            pltpu.VMEM((GCN_CH, GCHUNK), jnp.int32),
            pltpu.VMEM((GCHUNK, HID), jnp.float32),
            pltpu.VMEM_SHARED((NPAD, HID), jnp.float32),
            pltpu.SemaphoreType.DMA,
        ],
    )
    def k(hp_h, src_h, dst_h, zeros_h, out_h, src_v, dst_v, rows_v, acc_sh, sem):
        c = lax.axis_index("c")
        s = lax.axis_index("s")
        rows = NPAD // NSUB
        pltpu.sync_copy(zeros_h.at[pl.ds(0, rows)], acc_sh.at[pl.ds(s * rows, rows)])
        plsc.subcore_barrier()
        w = c * NSUB + s
        pltpu.sync_copy(src_h.at[w], src_v)
        pltpu.sync_copy(dst_h.at[w], dst_v)

        def body(j, _):
            pltpu.async_copy(hp_h.at[src_v.at[j]], rows_v, sem).wait()
            pltpu.sync_copy(rows_v, acc_sh.at[dst_v.at[j]], add=True)
            return 0

        lax.fori_loop(0, GCN_CH, body, 0)
        plsc.subcore_barrier()
        pltpu.sync_copy(acc_sh.at[pl.ds(s * rows, rows)],
                        out_h.at[c, pl.ds(s * rows, rows)])

    return k(hp, src3, dst3, zeros_rows)


def _sc_gather_edges(pairs):
    k_n = len(pairs)
    ns = [t.shape[0] for t, _ in pairs]

    @functools.partial(
        pl.kernel,
        out_type=[jax.ShapeDtypeStruct((GAT_E, 4), jnp.float32)] * k_n,
        mesh=_MESH,
        scratch_types=(
            [pltpu.VMEM((GAT_CH, GCHUNK), jnp.int32),
             pltpu.VMEM((GCHUNK, 4), jnp.float32),
             pltpu.SemaphoreType.DMA]
            + [pltpu.VMEM_SHARED((n_, 4), jnp.float32) for n_ in ns]
        ),
    )
    def k(*refs):
        tabs = [refs[2 * p] for p in range(k_n)]
        idxs = [refs[2 * p + 1] for p in range(k_n)]
        outs = list(refs[2 * k_n:2 * k_n + k_n])
        idx_v, ch_v, sem = refs[3 * k_n:3 * k_n + 3]
        tab_shs = list(refs[3 * k_n + 3:])
        c = lax.axis_index("c")
        s = lax.axis_index("s")
        w = c * NSUB + s
        for p in range(k_n):
            rows = ns[p] // NSUB
            pltpu.sync_copy(tabs[p].at[pl.ds(s * rows, rows)],
                            tab_shs[p].at[pl.ds(s * rows, rows)])
        plsc.subcore_barrier()
        for p in range(k_n):
            pltpu.sync_copy(idxs[p].at[w], idx_v)

            def body(j, _, p=p):
                pltpu.async_copy(tab_shs[p].at[idx_v.at[j]], ch_v, sem).wait()
                pltpu.sync_copy(
                    ch_v,
                    outs[p].at[pl.ds(w * (GAT_CH * GCHUNK) + j * GCHUNK,
                                     GCHUNK)])
                return 0

            lax.fori_loop(0, GAT_CH, body, 0)

    args = []
    for t, i3 in pairs:
        args += [t, i3]
    return k(*args)


def _sc_scatter_edges(triples, zeros_nodes):
    k_n = len(triples)
    npads = [t[2] for t in triples]

    @functools.partial(
        pl.kernel,
        out_type=[jax.ShapeDtypeStruct((NCORE, np_, 4), jnp.float32)
                  for np_ in npads],
        mesh=_MESH,
        scratch_types=(
            [pltpu.VMEM((GAT_CH, GCHUNK), jnp.int32),
             pltpu.VMEM((GCHUNK, 4), jnp.float32)]
            + [pltpu.VMEM_SHARED((np_, 4), jnp.float32) for np_ in npads]
        ),
    )
    def k(*refs):
        vals = [refs[2 * p] for p in range(k_n)]
        idxs = [refs[2 * p + 1] for p in range(k_n)]
        zeros_h = refs[2 * k_n]
        outs = list(refs[2 * k_n + 1:2 * k_n + 1 + k_n])
        idx_v, ch_v = refs[2 * k_n + 1 + k_n:2 * k_n + 3 + k_n]
        accs = list(refs[2 * k_n + 3 + k_n:])
        c = lax.axis_index("c")
        s = lax.axis_index("s")
        w = c * NSUB + s
        for p in range(k_n):
            rows = npads[p] // NSUB
            pltpu.sync_copy(zeros_h.at[pl.ds(0, rows)],
                            accs[p].at[pl.ds(s * rows, rows)])
        plsc.subcore_barrier()
        for p in range(k_n):
            pltpu.sync_copy(idxs[p].at[w], idx_v)

            def body(j, _, p=p):
                pltpu.sync_copy(
                    vals[p].at[pl.ds(w * (GAT_CH * GCHUNK) + j * GCHUNK,
                                     GCHUNK)], ch_v)
                pltpu.sync_copy(ch_v, accs[p].at[idx_v.at[j]], add=True)
                return 0

            lax.fori_loop(0, GAT_CH, body, 0)
        plsc.subcore_barrier()
        for p in range(k_n):
            rows = npads[p] // NSUB
            pltpu.sync_copy(accs[p].at[pl.ds(s * rows, rows)],
                            outs[p].at[c, pl.ds(s * rows, rows)])

    args = []
    for v, i3, _ in triples:
        args += [v, i3]
    args.append(zeros_nodes)
    return k(*args)


EBLK = GAT_CH * GCHUNK


def _ex_body(sg_ref, dg_ref, em_ref, ex_ref):
    e = sg_ref[...] + dg_ref[...]
    e = jnp.maximum(e, 0.2 * e)
    ex_ref[...] = jnp.exp(e - em_ref[...])


def _ex_stage(sg, dg, emax4):
    return pl.pallas_call(
        _ex_body,
        grid=(GAT_E // EBLK,),
        in_specs=[pl.BlockSpec((EBLK, 4), lambda i: (i, 0)),
                  pl.BlockSpec((EBLK, 4), lambda i: (i, 0)),
                  pl.BlockSpec((1, 4), lambda i: (0, 0))],
        out_specs=pl.BlockSpec((EBLK, 4), lambda i: (i, 0)),
        out_shape=jax.ShapeDtypeStruct((GAT_E, 4), jnp.float32),
    )(sg, dg, emax4)


def _alpha_body(ex_ref, den_ref, al_ref):
    al_ref[...] = ex_ref[...] / (den_ref[...] + 1e-16)


def _alpha_stage(ex, deng):
    return pl.pallas_call(
        _alpha_body,
        grid=(GAT_E // EBLK,),
        in_specs=[pl.BlockSpec((EBLK, 4), lambda i: (i, 0)),
                  pl.BlockSpec((EBLK, 4), lambda i: (i, 0))],
        out_specs=pl.BlockSpec((EBLK, 4), lambda i: (i, 0)),
        out_shape=jax.ShapeDtypeStruct((GAT_E, 4), jnp.float32),
    )(ex, deng)


def kernel(gene_nodes, drug_edges, disease_edges, gene_edges,
           gcn1_W, gcn1_b, gcn2_W, gcn2_b, drug_emb, dis_emb,
           gat_d_Wsrc, gat_d_Wdst, gat_d_asrc, gat_d_adst, gat_d_b,
           gat_s_Wsrc, gat_s_Wdst, gat_s_asrc, gat_s_adst, gat_s_b,
           fuse_W, fuse_b):
    v = _weights_prep(gat_d_Wsrc, gat_d_asrc, gat_d_Wdst, gat_d_adst,
                      gat_s_Wsrc, gat_s_asrc, gat_s_Wdst, gat_s_adst)
    vsd, vdd, vss, vds = v[:, 0:4], v[:, 4:8], v[:, 8:12], v[:, 12:16]
    vdst = jnp.concatenate([vdd, vds], axis=1)

    zeros_nodes = jnp.zeros((640, 4), jnp.float32)
    zeros_rows = jnp.zeros((640, HID), jnp.float32)
    ones_rows = jnp.concatenate(
        [jnp.ones((GCHUNK, 1), jnp.float32),
         jnp.zeros((GCHUNK, 3), jnp.float32)], axis=1)

    gsrc3 = gene_edges[0].reshape(32, GCN_CH, GCHUNK)
    gdst3 = gene_edges[1].reshape(32, GCN_CH, GCHUNK)
    npad = GAT_E - drug_edges.shape[1]
    padv = (jnp.arange(npad, dtype=jnp.int32) % 16)
    dsrc3 = jnp.concatenate([drug_edges[0], NUM_DRUGS + padv]
                            ).reshape(32, GAT_CH, GCHUNK)
    ddst3 = jnp.concatenate([drug_edges[1], N_GENE + padv]
                            ).reshape(32, GAT_CH, GCHUNK)
    ssrc3 = jnp.concatenate([disease_edges[0], N_GENE + padv]
                            ).reshape(32, GAT_CH, GCHUNK)
    sdst3 = jnp.concatenate([disease_edges[1], N_GENE + padv]
                            ).reshape(32, GAT_CH, GCHUNK)

    degp = _sc_deg(gdst3, ones_rows, zeros_nodes)
    h1, h1p, dinv = _h1_stage(gene_nodes, gcn1_W, degp[:, :N_GENE, 0:1])
    s1 = _sc_segsum(h1p, gsrc3, gdst3, zeros_rows)
    h2, h2p = _mid_stage(s1, h1, dinv, gcn1_b[None, :], gcn2_W)
    s2 = _sc_segsum(h2p, gsrc3, gdst3, zeros_rows)
    adst = _post_stage(s2, h2, dinv, gcn2_b[None, :], vdst)

    asrc_d, cs_d = _table_stage(drug_emb, vsd)
    asrc_s, cs_s = _table_stage(dis_emb, vss)
    emax = _emax_stage(adst, asrc_d, asrc_s)

    pad_d = jnp.zeros((DPAD - NUM_DRUGS, 4), jnp.float32)
    pad_n = jnp.zeros((NPAD - N_GENE, 4), jnp.float32)
    asrc_d_pad = jnp.concatenate([asrc_d, pad_d])
    asrc_s_pad = jnp.concatenate([asrc_s[:N_GENE], pad_n])
    adst_d_pad = jnp.concatenate([adst[:, :4], pad_n])
    adst_s_pad = jnp.concatenate([adst[:, 4:], pad_n])

    sg_d, dg_d, sg_s, dg_s = _sc_gather_edges(
        [(asrc_d_pad, dsrc3), (adst_d_pad, ddst3),
         (asrc_s_pad, ssrc3), (adst_s_pad, sdst3)])
    ex_d = _ex_stage(sg_d, dg_d, emax[:, :4])
    ex_s = _ex_stage(sg_s, dg_s, emax[:, 4:])
    denp_d, denp_s = _sc_scatter_edges(
        [(ex_d, ddst3, NPAD), (ex_s, sdst3, NPAD)], zeros_nodes)
    den_d = _psum_stage(denp_d)
    den_s = _psum_stage(denp_s)
    deng_d, deng_s = _sc_gather_edges([(den_d, ddst3), (den_s, sdst3)])
    al_d = _alpha_stage(ex_d, deng_d)
    al_s = _alpha_stage(ex_s, deng_s)
    wp_d, wp_s = _sc_scatter_edges(
        [(al_d, dsrc3, DPAD), (al_s, ssrc3, NPAD)], zeros_nodes)

    u_d = _u_stage(wp_d[:, :NUM_DRUGS, :], drug_emb)
    u_s = _u_stage(wp_s[:, :N_GENE, :], dis_emb[:N_GENE])

    return _final_stage(u_d, u_s, gat_d_Wsrc, gat_s_Wsrc, gat_d_b, gat_s_b,
                        cs_d[:, 0, :], cs_s[:, 0, :], fuse_W, fuse_b)
